# baseline (speedup 1.0000x reference)
import functools

import jax
import jax.numpy as jnp
from jax import lax
from jax.experimental import pallas as pl
from jax.experimental.pallas import tpu as pltpu
from jax.experimental.pallas import tpu_sc as plsc

N = 10000
E = 320000
D = 128
N_CLS = 40

NC = 2
NS = 16
NW = NC * NS
C = 128
NBUF = 2
GR = 8
NGRP0 = 16
NGRP1 = 4
NCH0 = GR * NGRP0
NCH1 = GR * NGRP1
TOTCH = NS * (NCH0 + NCH1)
EPAD = TOTCH * C
NPAD = 10112
ZROWS = NPAD // NS
ROWS_OUT = 624

ROW_BLK = 1000
GRID = N // ROW_BLK


def _zero_vmem_rows(ref, nrows, width):
    def body(i, carry):
        for j in range(width // 16):
            ref[i, pl.ds(j * 16, 16)] = jnp.zeros((16,), jnp.float32)
        return carry
    lax.fori_loop(0, nrows, body, 0)


def _segsum_body(g_hbm, src_hbm, dst_hbm, out_hbm, sidx_v, didx_v, rows_v,
                 acc, g0, g1, si0, si1, di0, di1):
    gsems = (g0, g1)
    ssems = (si0, si1)
    dsems = (di0, di1)
    cid = lax.axis_index("c")
    sid = lax.axis_index("s")
    zbase = sid * ZROWS

    _zero_vmem_rows(rows_v.at[0], C, D)
    done = 0
    while done < ZROWS:
        cnt = min(C, ZROWS - done)
        pltpu.sync_copy(rows_v.at[0, pl.ds(0, cnt)],
                        acc.at[pl.ds(zbase + done, cnt)])
        done += cnt

    plsc.subcore_barrier()

    def pipeline(base, ngrp):
        def issue_idx(grp, sp):
            off = pl.multiple_of(base + grp * GR, GR)
            pltpu.async_copy(src_hbm.at[pl.ds(off, GR)], sidx_v.at[sp],
                             ssems[sp])
            pltpu.async_copy(dst_hbm.at[pl.ds(off, GR)], didx_v.at[sp],
                             dsems[sp])

        def wait_idx(grp, sp):
            off = pl.multiple_of(base + grp * GR, GR)
            pltpu.make_async_copy(src_hbm.at[pl.ds(off, GR)],
                                  sidx_v.at[sp], ssems[sp]).wait()
            pltpu.make_async_copy(dst_hbm.at[pl.ds(off, GR)],
                                  didx_v.at[sp], dsems[sp]).wait()

        def issue_gather(sp, row, b):
            pltpu.async_copy(g_hbm.at[sidx_v.at[sp, row]], rows_v.at[b],
                             gsems[b])

        def wait_gather(sp, row, b):
            pltpu.make_async_copy(g_hbm.at[sidx_v.at[sp, row]], rows_v.at[b],
                                  gsems[b]).wait()

        def step(g_t, sp, q, has_next, has_next2):
            b = q % NBUF
            wait_gather(sp, q, b)
            pltpu.sync_copy(rows_v.at[b], acc.at[didx_v.at[sp, q]], add=True)
            if q == GR - NBUF and has_next:
                wait_idx(g_t + 1, sp ^ 1)
            if q < GR - NBUF:
                issue_gather(sp, q + NBUF, b)
            elif has_next:
                issue_gather(sp ^ 1, q + NBUF - GR, b)
            if q == GR - 1 and has_next2:
                issue_idx(g_t + 2, sp)

        issue_idx(0, 0)
        issue_idx(1, 1)
        wait_idx(0, 0)
        for j in range(NBUF):
            issue_gather(0, j, j)

        def main_body(gg, carry):
            g_t = gg * 2
            for sp in range(2):
                for q in range(GR):
                    step(g_t + sp, sp, q, True, True)
            return carry
        lax.fori_loop(0, (ngrp - 2) // 2, main_body, 0)

        for g in (ngrp - 2, ngrp - 1):
            for q in range(GR):
                step(g, g % 2, q, g + 1 <= ngrp - 1, g + 2 <= ngrp - 1)

    @pl.when(cid == 0)
    def _():
        pipeline(sid * NCH0, NGRP0)

    @pl.when(cid == 1)
    def _():
        pipeline(NS * NCH0 + sid * NCH1, NGRP1)

    plsc.subcore_barrier()

    pltpu.sync_copy(acc.at[pl.ds(sid * ROWS_OUT, ROWS_OUT)],
                    out_hbm.at[cid, pl.ds(sid * ROWS_OUT, ROWS_OUT)])

    @pl.when(sid == NS - 1)
    def _():
        tail = NS * ROWS_OUT
        pltpu.sync_copy(acc.at[pl.ds(tail, N - tail)],
                        out_hbm.at[cid, pl.ds(tail, N - tail)])


_segsum = functools.partial(
    pl.kernel,
    out_type=jax.ShapeDtypeStruct((NC, N, D), jnp.float32),
    mesh=plsc.VectorSubcoreMesh(core_axis_name="c", subcore_axis_name="s"),
    scratch_types=[
        pltpu.VMEM((2, GR, C), jnp.int32),
        pltpu.VMEM((2, GR, C), jnp.int32),
        pltpu.VMEM((NBUF, C, D), jnp.float32),
        pltpu.VMEM_SHARED((NPAD, D), jnp.float32),
        pltpu.SemaphoreType.DMA,
        pltpu.SemaphoreType.DMA,
        pltpu.SemaphoreType.DMA,
        pltpu.SemaphoreType.DMA,
        pltpu.SemaphoreType.DMA,
        pltpu.SemaphoreType.DMA,
    ],
)(_segsum_body)


def _proj0_body(x_ref, wn_ref, pdeg_ref, g_ref, inv_ref):
    g_ref[...] = jnp.dot(x_ref[...], wn_ref[...],
                         preferred_element_type=jnp.float32)
    d = pdeg_ref[0, :, 0:16] + pdeg_ref[1, :, 0:16]
    inv_ref[...] = 1.0 / jnp.maximum(d, 1.0)


def _proj0(x, wn0, pdeg):
    return pl.pallas_call(
        _proj0_body,
        grid=(GRID,),
        in_specs=[
            pl.BlockSpec((ROW_BLK, D), lambda i: (i, 0)),
            pl.BlockSpec((D, D), lambda i: (0, 0)),
            pl.BlockSpec((NC, ROW_BLK, D), lambda i: (0, i, 0)),
        ],
        out_specs=[
            pl.BlockSpec((ROW_BLK, D), lambda i: (i, 0)),
            pl.BlockSpec((ROW_BLK, 16), lambda i: (i, 0)),
        ],
        out_shape=[
            jax.ShapeDtypeStruct((N, D), jnp.float32),
            jax.ShapeDtypeStruct((N, 16), jnp.float32),
        ],
    )(x, wn0, pdeg)


def _combine_body(h_ref, p_ref, inv_ref, ws_ref, b_ref, wn_ref, h_out, g_out):
    agg = (p_ref[0] + p_ref[1]) * inv_ref[:, 0:1]
    t = jnp.dot(h_ref[...], ws_ref[...], preferred_element_type=jnp.float32)
    t = jnp.maximum(t + agg + b_ref[...], 0.0)
    h_out[...] = t
    g_out[...] = jnp.dot(t, wn_ref[...], preferred_element_type=jnp.float32)


def _combine(h, p, inv16, ws, b, wn_next):
    return pl.pallas_call(
        _combine_body,
        grid=(GRID,),
        in_specs=[
            pl.BlockSpec((ROW_BLK, D), lambda i: (i, 0)),
            pl.BlockSpec((NC, ROW_BLK, D), lambda i: (0, i, 0)),
            pl.BlockSpec((ROW_BLK, 16), lambda i: (i, 0)),
            pl.BlockSpec((D, D), lambda i: (0, 0)),
            pl.BlockSpec((1, D), lambda i: (0, 0)),
            pl.BlockSpec((D, D), lambda i: (0, 0)),
        ],
        out_specs=[
            pl.BlockSpec((ROW_BLK, D), lambda i: (i, 0)),
            pl.BlockSpec((ROW_BLK, D), lambda i: (i, 0)),
        ],
        out_shape=[
            jax.ShapeDtypeStruct((N, D), jnp.float32),
            jax.ShapeDtypeStruct((N, D), jnp.float32),
        ],
    )(h, p, inv16, ws, b.reshape(1, D), wn_next)


def _combine_last_body(h_ref, p_ref, inv_ref, ws_ref, b_ref, psum_out):
    agg = (p_ref[0] + p_ref[1]) * inv_ref[:, 0:1]
    t = jnp.dot(h_ref[...], ws_ref[...], preferred_element_type=jnp.float32)
    t = jnp.maximum(t + agg + b_ref[...], 0.0)
    psum_out[...] = jnp.sum(t.reshape(8, ROW_BLK // 8, D), axis=1)[None]


def _combine_last(h, p, inv16, ws, b):
    return pl.pallas_call(
        _combine_last_body,
        grid=(GRID,),
        in_specs=[
            pl.BlockSpec((ROW_BLK, D), lambda i: (i, 0)),
            pl.BlockSpec((NC, ROW_BLK, D), lambda i: (0, i, 0)),
            pl.BlockSpec((ROW_BLK, 16), lambda i: (i, 0)),
            pl.BlockSpec((D, D), lambda i: (0, 0)),
            pl.BlockSpec((1, D), lambda i: (0, 0)),
        ],
        out_specs=pl.BlockSpec((1, 8, D), lambda i: (i, 0, 0)),
        out_shape=jax.ShapeDtypeStruct((GRID, 8, D), jnp.float32),
    )(h, p, inv16, ws, b.reshape(1, D))


def _head_body(ps_ref, w1_ref, b1_ref, w2_ref, b2_ref, out_ref):
    m = jnp.sum(ps_ref[...], axis=(0, 1)).reshape(1, D) * (1.0 / N)
    y = jnp.dot(m, w1_ref[...], preferred_element_type=jnp.float32) + b1_ref[...]
    y = jnp.where(y > 0, y, jnp.exp(y) - 1.0)
    z = jnp.dot(y, w2_ref[...], preferred_element_type=jnp.float32) + b2_ref[...]
    mx = jnp.max(z, axis=0, keepdims=True)
    e = z - mx
    out_ref[...] = e - jnp.log(jnp.sum(jnp.exp(e), axis=0, keepdims=True))


def _head(psums, w1, b1, w2, b2):
    return pl.pallas_call(
        _head_body,
        out_shape=jax.ShapeDtypeStruct((1, N_CLS), jnp.float32),
    )(psums, w1, b1.reshape(1, D), w2, b2.reshape(1, N_CLS))


def kernel(x, edge_index, Ws0, Wn0, b0, Ws1, Wn1, b1, Ws2, Wn2, b2,
           Ws3, Wn3, b3, W_fc1, b_fc1, W_fc2, b_fc2):
    src = edge_index[0].astype(jnp.int32)
    dst = edge_index[1].astype(jnp.int32)
    pad = EPAD - E
    src2 = jnp.concatenate([src, jnp.zeros((pad,), jnp.int32)]).reshape(TOTCH, C)
    dst2 = jnp.concatenate([dst, jnp.full((pad,), N, jnp.int32)]).reshape(TOTCH, C)

    ones_tab = jnp.ones((N, D), jnp.float32)
    zeros2 = jnp.zeros((TOTCH, C), jnp.int32)
    pdeg = _segsum(ones_tab, zeros2, dst2)
    g, inv16 = _proj0(x, Wn0, pdeg)

    h = x
    for (ws, b, wn_next) in [(Ws0, b0, Wn1), (Ws1, b1, Wn2), (Ws2, b2, Wn3)]:
        p = _segsum(g, src2, dst2)
        h, g = _combine(h, p, inv16, ws, b, wn_next)
    p = _segsum(g, src2, dst2)
    psums = _combine_last(h, p, inv16, Ws3, b3)

    return _head(psums, W_fc1, b_fc1, W_fc2, b_fc2)


# speedup vs baseline: 6.0687x; 6.0687x over previous
import functools

import jax
import jax.numpy as jnp
from jax import lax
from jax.experimental import pallas as pl
from jax.experimental.pallas import tpu as pltpu
from jax.experimental.pallas import tpu_sc as plsc

N = 10000
E = 320000
D = 128
N_CLS = 40

NC = 2
NS = 16
NW = NC * NS
C = 128
NBUF = 2
GR = 8
NGRP0 = 20
NCH0 = GR * NGRP0
TOTCH = NS * NCH0
EPAD = TOTCH * C
NPAD = 10112
ZROWS = NPAD // NS
ROWS_OUT = 624

ROW_BLK = 1000
GRID = N // ROW_BLK


def _zero_vmem_rows(ref, nrows, width):
    def body(i, carry):
        for j in range(width // 16):
            ref[i, pl.ds(j * 16, 16)] = jnp.zeros((16,), jnp.float32)
        return carry
    lax.fori_loop(0, nrows, body, 0)


def _segsum_body(g_hbm, src_hbm, dst_hbm, out_hbm, sidx_v, didx_v, rows_v,
                 acc, g0, g1, si0, si1, di0, di1):
    gsems = (g0, g1)
    ssems = (si0, si1)
    dsems = (di0, di1)
    cid = lax.axis_index("c")
    sid = lax.axis_index("s")
    zbase = sid * ZROWS

    _zero_vmem_rows(rows_v.at[0], C, D)
    done = 0
    while done < ZROWS:
        cnt = min(C, ZROWS - done)
        pltpu.sync_copy(rows_v.at[0, pl.ds(0, cnt)],
                        acc.at[pl.ds(zbase + done, cnt)])
        done += cnt

    plsc.subcore_barrier()

    def pipeline(base, ngrp):
        def issue_idx(grp, sp):
            off = pl.multiple_of(base + grp * GR, GR)
            pltpu.async_copy(src_hbm.at[pl.ds(off, GR)], sidx_v.at[sp],
                             ssems[sp])
            pltpu.async_copy(dst_hbm.at[pl.ds(off, GR)], didx_v.at[sp],
                             dsems[sp])

        def wait_idx(grp, sp):
            off = pl.multiple_of(base + grp * GR, GR)
            pltpu.make_async_copy(src_hbm.at[pl.ds(off, GR)],
                                  sidx_v.at[sp], ssems[sp]).wait()
            pltpu.make_async_copy(dst_hbm.at[pl.ds(off, GR)],
                                  didx_v.at[sp], dsems[sp]).wait()

        def issue_gather(sp, row, b):
            pltpu.async_copy(g_hbm.at[sidx_v.at[sp, row]], rows_v.at[b],
                             gsems[b])

        def wait_gather(sp, row, b):
            pltpu.make_async_copy(g_hbm.at[sidx_v.at[sp, row]], rows_v.at[b],
                                  gsems[b]).wait()

        def step(g_t, sp, q, has_next, has_next2):
            b = q % NBUF
            wait_gather(sp, q, b)
            pltpu.sync_copy(rows_v.at[b], acc.at[didx_v.at[sp, q]], add=True)
            if q == GR - NBUF and has_next:
                wait_idx(g_t + 1, sp ^ 1)
            if q < GR - NBUF:
                issue_gather(sp, q + NBUF, b)
            elif has_next:
                issue_gather(sp ^ 1, q + NBUF - GR, b)
            if q == GR - 1 and has_next2:
                issue_idx(g_t + 2, sp)

        issue_idx(0, 0)
        issue_idx(1, 1)
        wait_idx(0, 0)
        for j in range(NBUF):
            issue_gather(0, j, j)

        def main_body(gg, carry):
            g_t = gg * 2
            for sp in range(2):
                for q in range(GR):
                    step(g_t + sp, sp, q, True, True)
            return carry
        lax.fori_loop(0, (ngrp - 2) // 2, main_body, 0)

        for g in (ngrp - 2, ngrp - 1):
            for q in range(GR):
                step(g, g % 2, q, g + 1 <= ngrp - 1, g + 2 <= ngrp - 1)

    @pl.when(cid == 0)
    def _():
        pipeline(sid * NCH0, NGRP0)

    plsc.subcore_barrier()

    pltpu.sync_copy(acc.at[pl.ds(sid * ROWS_OUT, ROWS_OUT)],
                    out_hbm.at[cid, pl.ds(sid * ROWS_OUT, ROWS_OUT)])

    @pl.when(sid == NS - 1)
    def _():
        tail = NS * ROWS_OUT
        pltpu.sync_copy(acc.at[pl.ds(tail, N - tail)],
                        out_hbm.at[cid, pl.ds(tail, N - tail)])


_segsum = functools.partial(
    pl.kernel,
    out_type=jax.ShapeDtypeStruct((NC, N, D), jnp.float32),
    mesh=plsc.VectorSubcoreMesh(core_axis_name="c", subcore_axis_name="s"),
    scratch_types=[
        pltpu.VMEM((2, GR, C), jnp.int32),
        pltpu.VMEM((2, GR, C), jnp.int32),
        pltpu.VMEM((NBUF, C, D), jnp.float32),
        pltpu.VMEM_SHARED((NPAD, D), jnp.float32),
        pltpu.SemaphoreType.DMA,
        pltpu.SemaphoreType.DMA,
        pltpu.SemaphoreType.DMA,
        pltpu.SemaphoreType.DMA,
        pltpu.SemaphoreType.DMA,
        pltpu.SemaphoreType.DMA,
    ],
)(_segsum_body)


def _proj0_body(x_ref, wn_ref, pdeg_ref, g_ref, inv_ref):
    g_ref[...] = jnp.dot(x_ref[...], wn_ref[...],
                         preferred_element_type=jnp.float32)
    d = pdeg_ref[0, :, 0:16] + pdeg_ref[1, :, 0:16]
    inv_ref[...] = 1.0 / jnp.maximum(d, 1.0)


def _proj0(x, wn0, pdeg):
    return pl.pallas_call(
        _proj0_body,
        grid=(GRID,),
        in_specs=[
            pl.BlockSpec((ROW_BLK, D), lambda i: (i, 0)),
            pl.BlockSpec((D, D), lambda i: (0, 0)),
            pl.BlockSpec((NC, ROW_BLK, D), lambda i: (0, i, 0)),
        ],
        out_specs=[
            pl.BlockSpec((ROW_BLK, D), lambda i: (i, 0)),
            pl.BlockSpec((ROW_BLK, 16), lambda i: (i, 0)),
        ],
        out_shape=[
            jax.ShapeDtypeStruct((N, D), jnp.float32),
            jax.ShapeDtypeStruct((N, 16), jnp.float32),
        ],
    )(x, wn0, pdeg)


def _combine_body(h_ref, p_ref, inv_ref, ws_ref, b_ref, wn_ref, h_out, g_out):
    agg = (p_ref[0] + p_ref[1]) * inv_ref[:, 0:1]
    t = jnp.dot(h_ref[...], ws_ref[...], preferred_element_type=jnp.float32)
    t = jnp.maximum(t + agg + b_ref[...], 0.0)
    h_out[...] = t
    g_out[...] = jnp.dot(t, wn_ref[...], preferred_element_type=jnp.float32)


def _combine(h, p, inv16, ws, b, wn_next):
    return pl.pallas_call(
        _combine_body,
        grid=(GRID,),
        in_specs=[
            pl.BlockSpec((ROW_BLK, D), lambda i: (i, 0)),
            pl.BlockSpec((NC, ROW_BLK, D), lambda i: (0, i, 0)),
            pl.BlockSpec((ROW_BLK, 16), lambda i: (i, 0)),
            pl.BlockSpec((D, D), lambda i: (0, 0)),
            pl.BlockSpec((1, D), lambda i: (0, 0)),
            pl.BlockSpec((D, D), lambda i: (0, 0)),
        ],
        out_specs=[
            pl.BlockSpec((ROW_BLK, D), lambda i: (i, 0)),
            pl.BlockSpec((ROW_BLK, D), lambda i: (i, 0)),
        ],
        out_shape=[
            jax.ShapeDtypeStruct((N, D), jnp.float32),
            jax.ShapeDtypeStruct((N, D), jnp.float32),
        ],
    )(h, p, inv16, ws, b.reshape(1, D), wn_next)


def _combine_last_body(h_ref, p_ref, inv_ref, ws_ref, b_ref, psum_out):
    agg = (p_ref[0] + p_ref[1]) * inv_ref[:, 0:1]
    t = jnp.dot(h_ref[...], ws_ref[...], preferred_element_type=jnp.float32)
    t = jnp.maximum(t + agg + b_ref[...], 0.0)
    psum_out[...] = jnp.sum(t.reshape(8, ROW_BLK // 8, D), axis=1)[None]


def _combine_last(h, p, inv16, ws, b):
    return pl.pallas_call(
        _combine_last_body,
        grid=(GRID,),
        in_specs=[
            pl.BlockSpec((ROW_BLK, D), lambda i: (i, 0)),
            pl.BlockSpec((NC, ROW_BLK, D), lambda i: (0, i, 0)),
            pl.BlockSpec((ROW_BLK, 16), lambda i: (i, 0)),
            pl.BlockSpec((D, D), lambda i: (0, 0)),
            pl.BlockSpec((1, D), lambda i: (0, 0)),
        ],
        out_specs=pl.BlockSpec((1, 8, D), lambda i: (i, 0, 0)),
        out_shape=jax.ShapeDtypeStruct((GRID, 8, D), jnp.float32),
    )(h, p, inv16, ws, b.reshape(1, D))


def _head_body(ps_ref, w1_ref, b1_ref, w2_ref, b2_ref, out_ref):
    m = jnp.sum(ps_ref[...], axis=(0, 1)).reshape(1, D) * (1.0 / N)
    y = jnp.dot(m, w1_ref[...], preferred_element_type=jnp.float32) + b1_ref[...]
    y = jnp.where(y > 0, y, jnp.exp(y) - 1.0)
    z = jnp.dot(y, w2_ref[...], preferred_element_type=jnp.float32) + b2_ref[...]
    mx = jnp.max(z, axis=0, keepdims=True)
    e = z - mx
    out_ref[...] = e - jnp.log(jnp.sum(jnp.exp(e), axis=0, keepdims=True))


def _head(psums, w1, b1, w2, b2):
    return pl.pallas_call(
        _head_body,
        out_shape=jax.ShapeDtypeStruct((1, N_CLS), jnp.float32),
    )(psums, w1, b1.reshape(1, D), w2, b2.reshape(1, N_CLS))


def kernel(x, edge_index, Ws0, Wn0, b0, Ws1, Wn1, b1, Ws2, Wn2, b2,
           Ws3, Wn3, b3, W_fc1, b_fc1, W_fc2, b_fc2):
    src = edge_index[0].astype(jnp.int32)
    dst = edge_index[1].astype(jnp.int32)
    pad = EPAD - E
    src2 = jnp.concatenate([src, jnp.zeros((pad,), jnp.int32)]).reshape(TOTCH, C)
    dst2 = jnp.concatenate([dst, jnp.full((pad,), N, jnp.int32)]).reshape(TOTCH, C)

    ones_tab = jnp.ones((N, D), jnp.float32)
    seq2 = (jnp.arange(EPAD, dtype=jnp.int32) % N).reshape(TOTCH, C)
    pdeg = _segsum(ones_tab, seq2, dst2)
    g, inv16 = _proj0(x, Wn0, pdeg)

    h = x
    for (ws, b, wn_next) in [(Ws0, b0, Wn1), (Ws1, b1, Wn2), (Ws2, b2, Wn3)]:
        p = _segsum(g, src2, dst2)
        h, g = _combine(h, p, inv16, ws, b, wn_next)
    p = _segsum(g, src2, dst2)
    psums = _combine_last(h, p, inv16, Ws3, b3)

    return _head(psums, W_fc1, b_fc1, W_fc2, b_fc2)


# speedup vs baseline: 6.3489x; 1.0462x over previous
import functools

import jax
import jax.numpy as jnp
from jax import lax
from jax.experimental import pallas as pl
from jax.experimental.pallas import tpu as pltpu
from jax.experimental.pallas import tpu_sc as plsc

N = 10000
E = 320000
D = 128
N_CLS = 40

NC = 2
NS = 16
NW = NC * NS
C = 128
NCH0 = 80
NCH1 = 80
NCHM = max(NCH0, NCH1)
TOTCH = NS * (NCH0 + NCH1)
EPAD = TOTCH * C
NPAD = 10112
ZROWS = NPAD // NS
ROWS_OUT = 624

ROW_BLK = 1000
GRID = N // ROW_BLK


def _zero_vmem_rows(ref, nrows, width):
    def body(i, carry):
        for j in range(width // 16):
            ref[i, pl.ds(j * 16, 16)] = jnp.zeros((16,), jnp.float32)
        return carry
    lax.fori_loop(0, nrows, body, 0)


def _segsum_body(g_hbm, src_hbm, dst_hbm, out_hbm, src_v, dst_v, rows_v,
                 acc, sem):
    cid = lax.axis_index("c")
    sid = lax.axis_index("s")
    zbase = sid * ZROWS

    _zero_vmem_rows(rows_v, C, D)
    done = 0
    while done < ZROWS:
        cnt = min(C, ZROWS - done)
        pltpu.sync_copy(rows_v.at[pl.ds(0, cnt)],
                        acc.at[pl.ds(zbase + done, cnt)])
        done += cnt

    plsc.subcore_barrier()

    def run(base, nch):
        pltpu.sync_copy(src_hbm.at[pl.ds(base, nch)], src_v.at[pl.ds(0, nch)])
        pltpu.sync_copy(dst_hbm.at[pl.ds(base, nch)], dst_v.at[pl.ds(0, nch)])

        def body(j, carry):
            pltpu.async_copy(g_hbm.at[src_v.at[j]], rows_v, sem).wait()
            pltpu.sync_copy(rows_v, acc.at[dst_v.at[j]], add=True)
            return carry
        lax.fori_loop(0, nch, body, 0)

    @pl.when(cid == 0)
    def _():
        run(pl.multiple_of(sid * NCH0, 8), NCH0)

    @pl.when(cid == 1)
    def _():
        run(pl.multiple_of(NS * NCH0 + sid * NCH1, 8), NCH1)

    plsc.subcore_barrier()

    pltpu.sync_copy(acc.at[pl.ds(sid * ROWS_OUT, ROWS_OUT)],
                    out_hbm.at[cid, pl.ds(sid * ROWS_OUT, ROWS_OUT)])

    @pl.when(sid == NS - 1)
    def _():
        tail = NS * ROWS_OUT
        pltpu.sync_copy(acc.at[pl.ds(tail, N - tail)],
                        out_hbm.at[cid, pl.ds(tail, N - tail)])


_segsum = functools.partial(
    pl.kernel,
    out_type=jax.ShapeDtypeStruct((NC, N, D), jnp.float32),
    mesh=plsc.VectorSubcoreMesh(core_axis_name="c", subcore_axis_name="s"),
    scratch_types=[
        pltpu.VMEM((NCHM, C), jnp.int32),
        pltpu.VMEM((NCHM, C), jnp.int32),
        pltpu.VMEM((C, D), jnp.float32),
        pltpu.VMEM_SHARED((NPAD, D), jnp.float32),
        pltpu.SemaphoreType.DMA,
    ],
)(_segsum_body)


def _proj0_body(x_ref, wn_ref, pdeg_ref, g_ref, inv_ref):
    g_ref[...] = jnp.dot(x_ref[...], wn_ref[...],
                         preferred_element_type=jnp.float32)
    d = pdeg_ref[0, :, 0:16] + pdeg_ref[1, :, 0:16]
    inv_ref[...] = 1.0 / jnp.maximum(d, 1.0)


def _proj0(x, wn0, pdeg):
    return pl.pallas_call(
        _proj0_body,
        grid=(GRID,),
        in_specs=[
            pl.BlockSpec((ROW_BLK, D), lambda i: (i, 0)),
            pl.BlockSpec((D, D), lambda i: (0, 0)),
            pl.BlockSpec((NC, ROW_BLK, D), lambda i: (0, i, 0)),
        ],
        out_specs=[
            pl.BlockSpec((ROW_BLK, D), lambda i: (i, 0)),
            pl.BlockSpec((ROW_BLK, 16), lambda i: (i, 0)),
        ],
        out_shape=[
            jax.ShapeDtypeStruct((N, D), jnp.float32),
            jax.ShapeDtypeStruct((N, 16), jnp.float32),
        ],
    )(x, wn0, pdeg)


def _combine_body(h_ref, p_ref, inv_ref, ws_ref, b_ref, wn_ref, h_out, g_out):
    agg = (p_ref[0] + p_ref[1]) * inv_ref[:, 0:1]
    t = jnp.dot(h_ref[...], ws_ref[...], preferred_element_type=jnp.float32)
    t = jnp.maximum(t + agg + b_ref[...], 0.0)
    h_out[...] = t
    g_out[...] = jnp.dot(t, wn_ref[...], preferred_element_type=jnp.float32)


def _combine(h, p, inv16, ws, b, wn_next):
    return pl.pallas_call(
        _combine_body,
        grid=(GRID,),
        in_specs=[
            pl.BlockSpec((ROW_BLK, D), lambda i: (i, 0)),
            pl.BlockSpec((NC, ROW_BLK, D), lambda i: (0, i, 0)),
            pl.BlockSpec((ROW_BLK, 16), lambda i: (i, 0)),
            pl.BlockSpec((D, D), lambda i: (0, 0)),
            pl.BlockSpec((1, D), lambda i: (0, 0)),
            pl.BlockSpec((D, D), lambda i: (0, 0)),
        ],
        out_specs=[
            pl.BlockSpec((ROW_BLK, D), lambda i: (i, 0)),
            pl.BlockSpec((ROW_BLK, D), lambda i: (i, 0)),
        ],
        out_shape=[
            jax.ShapeDtypeStruct((N, D), jnp.float32),
            jax.ShapeDtypeStruct((N, D), jnp.float32),
        ],
    )(h, p, inv16, ws, b.reshape(1, D), wn_next)


def _combine_last_body(h_ref, p_ref, inv_ref, ws_ref, b_ref, psum_out):
    agg = (p_ref[0] + p_ref[1]) * inv_ref[:, 0:1]
    t = jnp.dot(h_ref[...], ws_ref[...], preferred_element_type=jnp.float32)
    t = jnp.maximum(t + agg + b_ref[...], 0.0)
    psum_out[...] = jnp.sum(t.reshape(8, ROW_BLK // 8, D), axis=1)[None]


def _combine_last(h, p, inv16, ws, b):
    return pl.pallas_call(
        _combine_last_body,
        grid=(GRID,),
        in_specs=[
            pl.BlockSpec((ROW_BLK, D), lambda i: (i, 0)),
            pl.BlockSpec((NC, ROW_BLK, D), lambda i: (0, i, 0)),
            pl.BlockSpec((ROW_BLK, 16), lambda i: (i, 0)),
            pl.BlockSpec((D, D), lambda i: (0, 0)),
            pl.BlockSpec((1, D), lambda i: (0, 0)),
        ],
        out_specs=pl.BlockSpec((1, 8, D), lambda i: (i, 0, 0)),
        out_shape=jax.ShapeDtypeStruct((GRID, 8, D), jnp.float32),
    )(h, p, inv16, ws, b.reshape(1, D))


def _head_body(ps_ref, w1_ref, b1_ref, w2_ref, b2_ref, out_ref):
    m = jnp.sum(ps_ref[...], axis=(0, 1)).reshape(1, D) * (1.0 / N)
    y = jnp.dot(m, w1_ref[...], preferred_element_type=jnp.float32) + b1_ref[...]
    y = jnp.where(y > 0, y, jnp.exp(y) - 1.0)
    z = jnp.dot(y, w2_ref[...], preferred_element_type=jnp.float32) + b2_ref[...]
    mx = jnp.max(z, axis=0, keepdims=True)
    e = z - mx
    out_ref[...] = e - jnp.log(jnp.sum(jnp.exp(e), axis=0, keepdims=True))


def _head(psums, w1, b1, w2, b2):
    return pl.pallas_call(
        _head_body,
        out_shape=jax.ShapeDtypeStruct((1, N_CLS), jnp.float32),
    )(psums, w1, b1.reshape(1, D), w2, b2.reshape(1, N_CLS))


def kernel(x, edge_index, Ws0, Wn0, b0, Ws1, Wn1, b1, Ws2, Wn2, b2,
           Ws3, Wn3, b3, W_fc1, b_fc1, W_fc2, b_fc2):
    src = edge_index[0].astype(jnp.int32)
    dst = edge_index[1].astype(jnp.int32)
    pad = EPAD - E
    src2 = jnp.concatenate([src, jnp.zeros((pad,), jnp.int32)]).reshape(TOTCH, C)
    dst2 = jnp.concatenate([dst, jnp.full((pad,), N, jnp.int32)]).reshape(TOTCH, C)

    ones_tab = jnp.ones((N, D), jnp.float32)
    seq2 = (jnp.arange(EPAD, dtype=jnp.int32) % N).reshape(TOTCH, C)
    pdeg = _segsum(ones_tab, seq2, dst2)
    g, inv16 = _proj0(x, Wn0, pdeg)

    h = x
    for (ws, b, wn_next) in [(Ws0, b0, Wn1), (Ws1, b1, Wn2), (Ws2, b2, Wn3)]:
        p = _segsum(g, src2, dst2)
        h, g = _combine(h, p, inv16, ws, b, wn_next)
    p = _segsum(g, src2, dst2)
    psums = _combine_last(h, p, inv16, Ws3, b3)

    return _head(psums, W_fc1, b_fc1, W_fc2, b_fc2)


# speedup vs baseline: 9.8906x; 1.5579x over previous
import functools

import jax
import jax.numpy as jnp
from jax import lax
from jax.experimental import pallas as pl
from jax.experimental.pallas import tpu as pltpu
from jax.experimental.pallas import tpu_sc as plsc

N = 10000
E = 320000
D = 128
N_CLS = 40

NC = 2
NS = 16
NW = NC * NS
C = 128
NCH = -(-E // (NW * C))
EPAD = NW * NCH * C
NPAD = 10112
ZROWS = NPAD // NS
ROWS_OUT = 624

ROW_BLK = 1000
GRID = N // ROW_BLK


def _fill_vmem_rows(ref, nrows, width, value):
    def body(i, carry):
        for j in range(width // 16):
            ref[i, pl.ds(j * 16, 16)] = jnp.full((16,), value, jnp.float32)
        return carry
    lax.fori_loop(0, nrows, body, 0)


def _segsum_body(g_hbm, src_hbm, dst_hbm, out_hbm, src_v, dst_v, rows_v, acc, sem):
    cid = lax.axis_index("c")
    sid = lax.axis_index("s")
    wid = cid * NS + sid

    _fill_vmem_rows(rows_v, C, D, 0.0)
    base = sid * ZROWS
    done = 0
    while done < ZROWS:
        cnt = min(C, ZROWS - done)
        pltpu.sync_copy(rows_v.at[pl.ds(0, cnt)], acc.at[pl.ds(base + done, cnt)])
        done += cnt

    pltpu.sync_copy(src_hbm.at[wid], src_v)
    pltpu.sync_copy(dst_hbm.at[wid], dst_v)

    plsc.subcore_barrier()

    def body(j, carry):
        pltpu.async_copy(g_hbm.at[src_v.at[j]], rows_v, sem).wait()
        pltpu.sync_copy(rows_v, acc.at[dst_v.at[j]], add=True)
        return carry
    lax.fori_loop(0, NCH, body, 0)

    plsc.subcore_barrier()

    pltpu.sync_copy(acc.at[pl.ds(sid * ROWS_OUT, ROWS_OUT)],
                    out_hbm.at[cid, pl.ds(sid * ROWS_OUT, ROWS_OUT)])

    @pl.when(sid == NS - 1)
    def _():
        tail = NS * ROWS_OUT
        pltpu.sync_copy(acc.at[pl.ds(tail, N - tail)],
                        out_hbm.at[cid, pl.ds(tail, N - tail)])


_segsum = functools.partial(
    pl.kernel,
    out_type=jax.ShapeDtypeStruct((NC, N, D), jnp.float32),
    mesh=plsc.VectorSubcoreMesh(core_axis_name="c", subcore_axis_name="s"),
    scratch_types=[
        pltpu.VMEM((NCH, C), jnp.int32),
        pltpu.VMEM((NCH, C), jnp.int32),
        pltpu.VMEM((C, D), jnp.float32),
        pltpu.VMEM_SHARED((NPAD, D), jnp.float32),
        pltpu.SemaphoreType.DMA,
    ],
)(_segsum_body)


def _deg_body(dst_hbm, out_hbm, dst_v, ones_v, zero_v, acc):
    cid = lax.axis_index("c")
    sid = lax.axis_index("s")
    wid = cid * NS + sid

    _fill_vmem_rows(zero_v, C, D, 0.0)
    _fill_vmem_rows(ones_v, C, D, 1.0)
    base = sid * ZROWS
    done = 0
    while done < ZROWS:
        cnt = min(C, ZROWS - done)
        pltpu.sync_copy(zero_v.at[pl.ds(0, cnt)], acc.at[pl.ds(base + done, cnt)])
        done += cnt

    pltpu.sync_copy(dst_hbm.at[wid], dst_v)

    plsc.subcore_barrier()

    def body(j, carry):
        pltpu.sync_copy(ones_v, acc.at[dst_v.at[j]], add=True)
        return carry
    lax.fori_loop(0, NCH, body, 0)

    plsc.subcore_barrier()

    pltpu.sync_copy(acc.at[pl.ds(sid * ROWS_OUT, ROWS_OUT)],
                    out_hbm.at[cid, pl.ds(sid * ROWS_OUT, ROWS_OUT)])

    @pl.when(sid == NS - 1)
    def _():
        tail = NS * ROWS_OUT
        pltpu.sync_copy(acc.at[pl.ds(tail, N - tail)],
                        out_hbm.at[cid, pl.ds(tail, N - tail)])


_deg = functools.partial(
    pl.kernel,
    out_type=jax.ShapeDtypeStruct((NC, N, D), jnp.float32),
    mesh=plsc.VectorSubcoreMesh(core_axis_name="c", subcore_axis_name="s"),
    scratch_types=[
        pltpu.VMEM((NCH, C), jnp.int32),
        pltpu.VMEM((C, D), jnp.float32),
        pltpu.VMEM((C, D), jnp.float32),
        pltpu.VMEM_SHARED((NPAD, D), jnp.float32),
    ],
)(_deg_body)


def _proj0_body(x_ref, wn_ref, pdeg_ref, g_ref, inv_ref):
    g_ref[...] = jnp.dot(x_ref[...], wn_ref[...],
                         preferred_element_type=jnp.float32)
    d = pdeg_ref[0, :, 0:16] + pdeg_ref[1, :, 0:16]
    inv_ref[...] = 1.0 / jnp.maximum(d, 1.0)


def _proj0(x, wn0, pdeg):
    return pl.pallas_call(
        _proj0_body,
        grid=(GRID,),
        in_specs=[
            pl.BlockSpec((ROW_BLK, D), lambda i: (i, 0)),
            pl.BlockSpec((D, D), lambda i: (0, 0)),
            pl.BlockSpec((NC, ROW_BLK, D), lambda i: (0, i, 0)),
        ],
        out_specs=[
            pl.BlockSpec((ROW_BLK, D), lambda i: (i, 0)),
            pl.BlockSpec((ROW_BLK, 16), lambda i: (i, 0)),
        ],
        out_shape=[
            jax.ShapeDtypeStruct((N, D), jnp.float32),
            jax.ShapeDtypeStruct((N, 16), jnp.float32),
        ],
    )(x, wn0, pdeg)


def _combine_body(h_ref, p_ref, inv_ref, ws_ref, b_ref, wn_ref, h_out, g_out):
    agg = (p_ref[0] + p_ref[1]) * inv_ref[:, 0:1]
    t = jnp.dot(h_ref[...], ws_ref[...], preferred_element_type=jnp.float32)
    t = jnp.maximum(t + agg + b_ref[...], 0.0)
    h_out[...] = t
    g_out[...] = jnp.dot(t, wn_ref[...], preferred_element_type=jnp.float32)


def _combine(h, p, inv16, ws, b, wn_next):
    return pl.pallas_call(
        _combine_body,
        grid=(GRID,),
        in_specs=[
            pl.BlockSpec((ROW_BLK, D), lambda i: (i, 0)),
            pl.BlockSpec((NC, ROW_BLK, D), lambda i: (0, i, 0)),
            pl.BlockSpec((ROW_BLK, 16), lambda i: (i, 0)),
            pl.BlockSpec((D, D), lambda i: (0, 0)),
            pl.BlockSpec((1, D), lambda i: (0, 0)),
            pl.BlockSpec((D, D), lambda i: (0, 0)),
        ],
        out_specs=[
            pl.BlockSpec((ROW_BLK, D), lambda i: (i, 0)),
            pl.BlockSpec((ROW_BLK, D), lambda i: (i, 0)),
        ],
        out_shape=[
            jax.ShapeDtypeStruct((N, D), jnp.float32),
            jax.ShapeDtypeStruct((N, D), jnp.float32),
        ],
    )(h, p, inv16, ws, b.reshape(1, D), wn_next)


def _combine_last_body(h_ref, p_ref, inv_ref, ws_ref, b_ref, psum_out):
    agg = (p_ref[0] + p_ref[1]) * inv_ref[:, 0:1]
    t = jnp.dot(h_ref[...], ws_ref[...], preferred_element_type=jnp.float32)
    t = jnp.maximum(t + agg + b_ref[...], 0.0)
    psum_out[...] = jnp.sum(t.reshape(8, ROW_BLK // 8, D), axis=1)[None]


def _combine_last(h, p, inv16, ws, b):
    return pl.pallas_call(
        _combine_last_body,
        grid=(GRID,),
        in_specs=[
            pl.BlockSpec((ROW_BLK, D), lambda i: (i, 0)),
            pl.BlockSpec((NC, ROW_BLK, D), lambda i: (0, i, 0)),
            pl.BlockSpec((ROW_BLK, 16), lambda i: (i, 0)),
            pl.BlockSpec((D, D), lambda i: (0, 0)),
            pl.BlockSpec((1, D), lambda i: (0, 0)),
        ],
        out_specs=pl.BlockSpec((1, 8, D), lambda i: (i, 0, 0)),
        out_shape=jax.ShapeDtypeStruct((GRID, 8, D), jnp.float32),
    )(h, p, inv16, ws, b.reshape(1, D))


def _head_body(ps_ref, w1_ref, b1_ref, w2_ref, b2_ref, out_ref):
    m = jnp.sum(ps_ref[...], axis=(0, 1)).reshape(1, D) * (1.0 / N)
    y = jnp.dot(m, w1_ref[...], preferred_element_type=jnp.float32) + b1_ref[...]
    y = jnp.where(y > 0, y, jnp.exp(y) - 1.0)
    z = jnp.dot(y, w2_ref[...], preferred_element_type=jnp.float32) + b2_ref[...]
    mx = jnp.max(z, axis=0, keepdims=True)
    e = z - mx
    out_ref[...] = e - jnp.log(jnp.sum(jnp.exp(e), axis=0, keepdims=True))


def _head(psums, w1, b1, w2, b2):
    return pl.pallas_call(
        _head_body,
        out_shape=jax.ShapeDtypeStruct((1, N_CLS), jnp.float32),
    )(psums, w1, b1.reshape(1, D), w2, b2.reshape(1, N_CLS))


def kernel(x, edge_index, Ws0, Wn0, b0, Ws1, Wn1, b1, Ws2, Wn2, b2,
           Ws3, Wn3, b3, W_fc1, b_fc1, W_fc2, b_fc2):
    src = edge_index[0].astype(jnp.int32)
    dst = edge_index[1].astype(jnp.int32)
    pad = EPAD - E
    src3 = jnp.concatenate([src, jnp.zeros((pad,), jnp.int32)]).reshape(NW, NCH, C)
    dst3 = jnp.concatenate([dst, jnp.full((pad,), N, jnp.int32)]).reshape(NW, NCH, C)

    pdeg = _deg(dst3)
    g, inv16 = _proj0(x, Wn0, pdeg)

    h = x
    for (ws, b, wn_next) in [(Ws0, b0, Wn1), (Ws1, b1, Wn2), (Ws2, b2, Wn3)]:
        p = _segsum(g, src3, dst3)
        h, g = _combine(h, p, inv16, ws, b, wn_next)
    p = _segsum(g, src3, dst3)
    psums = _combine_last(h, p, inv16, Ws3, b3)

    return _head(psums, W_fc1, b_fc1, W_fc2, b_fc2)
